# baseline (speedup 1.0000x reference)
import jax
import jax.numpy as jnp
from jax import lax
from jax.experimental import pallas as pl
from jax.experimental.pallas import tpu as pltpu
from jax.experimental.pallas import tpu_sc as plsc

N = 10000
E = 320000
NP = 10240
F = 16
B = 128
NC = 2
NS = 16
NW = NC * NS
NCHUNK = E // B
CPW = 79
RPS = NP // NS

_mesh = plsc.VectorSubcoreMesh(core_axis_name="c", subcore_axis_name="s")
_sc_params = pltpu.CompilerParams(use_tc_tiling_on_sc=False)


def _sc_deg_body(dst_hbm, zeros_hbm, ones_hbm, out_hbm, ones_v, didx_v, sem, acc_sh):
    c = lax.axis_index("c")
    s = lax.axis_index("s")
    w = s * NC + c
    w79 = w * CPW
    base = jnp.minimum(w79, NCHUNK - CPW)
    o = w79 - base
    nj = jnp.minimum(CPW, NCHUNK - w79)
    pltpu.sync_copy(zeros_hbm.at[pl.ds(s * RPS, RPS)], acc_sh.at[pl.ds(s * RPS, RPS)])
    pltpu.sync_copy(ones_hbm, ones_v)
    pltpu.sync_copy(dst_hbm.at[pl.ds(base, CPW)], didx_v)
    plsc.subcore_barrier()

    def fire(j, carry):
        pltpu.async_copy(ones_v, acc_sh.at[didx_v.at[j + o]], sem, add=True)
        return carry

    lax.fori_loop(0, nj, fire, 0)

    def drain(j, carry):
        pltpu.make_async_copy(ones_v, acc_sh.at[didx_v.at[j + o]], sem).wait()
        return carry

    lax.fori_loop(0, nj, drain, 0)
    plsc.subcore_barrier()
    pltpu.sync_copy(acc_sh.at[pl.ds(s * RPS, RPS)], out_hbm.at[c, pl.ds(s * RPS, RPS)])


RING = 8
PF = 4


def _sc_prop_body(p_hbm, src_hbm, dst_hbm, zeros_hbm, out_hbm,
                  sidx_v, didx_v, rows_v, gsem, ssem, acc_sh):
    c = lax.axis_index("c")
    s = lax.axis_index("s")
    w = s * NC + c
    w79 = w * CPW
    base = jnp.minimum(w79, NCHUNK - CPW)
    o = w79 - base
    nj = jnp.minimum(CPW, NCHUNK - w79)
    pltpu.sync_copy(zeros_hbm.at[pl.ds(s * RPS, RPS)], acc_sh.at[pl.ds(s * RPS, RPS)])
    pltpu.sync_copy(src_hbm.at[pl.ds(base, CPW)], sidx_v)
    pltpu.sync_copy(dst_hbm.at[pl.ds(base, CPW)], didx_v)
    plsc.subcore_barrier()

    def prime(j, carry):
        @pl.when(j < nj)
        def _():
            pltpu.async_copy(p_hbm.at[sidx_v.at[j + o]], rows_v.at[j], gsem.at[j])
        return carry

    lax.fori_loop(0, PF, prime, 0)

    def body(j, carry):
        b = lax.rem(j, RING)
        bn = lax.rem(j + PF, RING)

        @pl.when(j >= RING - PF)
        def _():
            pltpu.make_async_copy(rows_v.at[bn],
                                  acc_sh.at[didx_v.at[j - (RING - PF) + o]],
                                  ssem.at[bn]).wait()

        @pl.when(j + PF < nj)
        def _():
            pltpu.async_copy(p_hbm.at[sidx_v.at[j + PF + o]], rows_v.at[bn], gsem.at[bn])

        pltpu.make_async_copy(p_hbm.at[sidx_v.at[j + o]], rows_v.at[b], gsem.at[b]).wait()
        pltpu.async_copy(rows_v.at[b], acc_sh.at[didx_v.at[j + o]], ssem.at[b], add=True)
        return carry

    lax.fori_loop(0, nj, body, 0)

    def drain(k, carry):
        j = nj - (RING - PF) + k

        @pl.when(j >= 0)
        def _():
            pltpu.make_async_copy(rows_v.at[lax.rem(j, RING)],
                                  acc_sh.at[didx_v.at[j + o]],
                                  ssem.at[lax.rem(j, RING)]).wait()
        return carry

    lax.fori_loop(0, RING - PF, drain, 0)
    plsc.subcore_barrier()
    pltpu.sync_copy(acc_sh.at[pl.ds(s * RPS, RPS)], out_hbm.at[c, pl.ds(s * RPS, RPS)])


_sc_deg = pl.kernel(
    _sc_deg_body,
    out_type=jax.ShapeDtypeStruct((NC, NP, F), jnp.float32),
    mesh=_mesh,
    compiler_params=_sc_params,
    scratch_types=[
        pltpu.VMEM((B, F), jnp.float32),
        pltpu.VMEM((CPW, B), jnp.int32),
        pltpu.SemaphoreType.DMA,
        pltpu.VMEM_SHARED((NP, F), jnp.float32),
    ],
)

_sc_prop = pl.kernel(
    _sc_prop_body,
    out_type=jax.ShapeDtypeStruct((NC, NP, F), jnp.float32),
    mesh=_mesh,
    compiler_params=_sc_params,
    scratch_types=[
        pltpu.VMEM((CPW, B), jnp.int32),
        pltpu.VMEM((CPW, B), jnp.int32),
        pltpu.VMEM((RING, B, F), jnp.float32),
        pltpu.SemaphoreType.DMA((RING,)),
        pltpu.SemaphoreType.DMA((RING,)),
        pltpu.VMEM_SHARED((NP, F), jnp.float32),
    ],
)


PK = N // 8
PKP = NP // 8


def _tc_cvt_body(ei_ref, s_ref, d_ref):
    s_ref[...] = ei_ref[0]
    d_ref[...] = ei_ref[1]


def _tc_cvt(edge_index):
    return pl.pallas_call(
        _tc_cvt_body,
        out_shape=[jax.ShapeDtypeStruct((E,), jnp.int32),
                   jax.ShapeDtypeStruct((E,), jnp.int32)],
    )(edge_index)


def _tc_mm_body(x_ref, w_ref, p0_ref):
    p0_ref[...] = jnp.dot(x_ref[...], w_ref[...],
                          preferred_element_type=jnp.float32)


def _tc_scale_body(p0_ref, degp_ref, p_ref, dinv_ref):
    deg = degp_ref[0, :PK] + degp_ref[1, :PK]
    dinv = lax.rsqrt(jnp.maximum(deg, 1.0))
    p_ref[...] = p0_ref[...] * dinv
    dinv_ref[...] = dinv


def _tc_b_body(a1p_ref, dinv_ref, q_ref):
    dinv = dinv_ref[...]
    a1 = (a1p_ref[0, :PK] + a1p_ref[1, :PK]) * dinv
    q_ref[...] = jnp.maximum(a1, 0.0) * dinv


def _tc_cm_body(a2p_ref, dinv_ref, m_ref):
    m_ref[...] = (a2p_ref[0, :PK] + a2p_ref[1, :PK]) * dinv_ref[...]


def _tc_mm2_body(m_ref, w2_ref, out_ref):
    out_ref[...] = jnp.dot(m_ref[...], w2_ref[...],
                           preferred_element_type=jnp.float32)


def _tc_mm(x, W1):
    return pl.pallas_call(
        _tc_mm_body,
        out_shape=jax.ShapeDtypeStruct((N, F), jnp.float32),
    )(x, W1)


def _tc_scale(p0, degp_pk):
    return pl.pallas_call(
        _tc_scale_body,
        out_shape=[
            jax.ShapeDtypeStruct((PK, 128), jnp.float32),
            jax.ShapeDtypeStruct((PK, 128), jnp.float32),
        ],
    )(p0, degp_pk)


def _tc_b(a1p_pk, dinv_pk):
    return pl.pallas_call(
        _tc_b_body,
        out_shape=jax.ShapeDtypeStruct((PK, 128), jnp.float32),
    )(a1p_pk, dinv_pk)


def _tc_cm(a2p_pk, dinv_pk):
    return pl.pallas_call(
        _tc_cm_body,
        out_shape=jax.ShapeDtypeStruct((PK, 128), jnp.float32),
    )(a2p_pk, dinv_pk)


def _tc_mm2(m, W2):
    return pl.pallas_call(
        _tc_mm2_body,
        out_shape=jax.ShapeDtypeStruct((N, 64), jnp.float32),
    )(m, W2)


@jax.jit
def kernel(x, edge_index, W1, W2):
    src1, dst1 = _tc_cvt(edge_index)
    src2 = src1.reshape(NCHUNK, B)
    dst2 = dst1.reshape(NCHUNK, B)
    zeros_np = jnp.zeros((NP, F), jnp.float32)
    ones_b = jnp.ones((B, F), jnp.float32)

    p0 = _tc_mm(x, W1)
    deg_parts = _sc_deg(dst2, zeros_np, ones_b)
    p_pk, dinv_pk = _tc_scale(p0.reshape(PK, 128), deg_parts.reshape(NC, PKP, 128))
    a1p = _sc_prop(p_pk.reshape(N, F), src2, dst2, zeros_np)
    q_pk = _tc_b(a1p.reshape(NC, PKP, 128), dinv_pk)
    a2p = _sc_prop(q_pk.reshape(N, F), src2, dst2, zeros_np)
    m_pk = _tc_cm(a2p.reshape(NC, PKP, 128), dinv_pk)
    return _tc_mm2(m_pk.reshape(N, F), W2)


# speedup vs baseline: 1.0792x; 1.0792x over previous
import jax
import jax.numpy as jnp
from jax import lax
from jax.experimental import pallas as pl
from jax.experimental.pallas import tpu as pltpu
from jax.experimental.pallas import tpu_sc as plsc

N = 10000
E = 320000
NP = 10240
F = 16
B = 128
NC = 2
NS = 16
NW = NC * NS
NCHUNK = E // B
CPW = 79
RPS = NP // NS

_mesh = plsc.VectorSubcoreMesh(core_axis_name="c", subcore_axis_name="s")
_sc_params = pltpu.CompilerParams(use_tc_tiling_on_sc=False)


def _sc_deg_body(dst_hbm, zeros_hbm, ones_hbm, out_hbm, ones_v, didx_v, sem, acc_sh):
    c = lax.axis_index("c")
    s = lax.axis_index("s")
    w = s * NC + c
    w79 = w * CPW
    base = jnp.minimum(w79, NCHUNK - CPW)
    o = w79 - base
    nj = jnp.minimum(CPW, NCHUNK - w79)
    pltpu.sync_copy(zeros_hbm.at[pl.ds(s * RPS, RPS)], acc_sh.at[pl.ds(s * RPS, RPS)])
    pltpu.sync_copy(ones_hbm, ones_v)
    pltpu.sync_copy(dst_hbm.at[pl.ds(base, CPW)], didx_v)
    plsc.subcore_barrier()

    def fire(j, carry):
        pltpu.async_copy(ones_v, acc_sh.at[didx_v.at[j + o]], sem, add=True)
        return carry

    lax.fori_loop(0, nj, fire, 0)

    def drain(j, carry):
        pltpu.make_async_copy(ones_v, acc_sh.at[didx_v.at[j + o]], sem).wait()
        return carry

    lax.fori_loop(0, nj, drain, 0)
    plsc.subcore_barrier()
    pltpu.sync_copy(acc_sh.at[pl.ds(s * RPS, RPS)], out_hbm.at[c, pl.ds(s * RPS, RPS)])


RING = 8
PF = 4


def _sc_prop_body(p_hbm, src_hbm, dst_hbm, zeros_hbm, out_hbm,
                  sidx_v, didx_v, rows_v, gsem, ssem, acc_sh):
    c = lax.axis_index("c")
    s = lax.axis_index("s")
    w = s * NC + c
    w79 = w * CPW
    base = jnp.minimum(w79, NCHUNK - CPW)
    o = w79 - base
    nj = jnp.minimum(CPW, NCHUNK - w79)
    pltpu.sync_copy(zeros_hbm.at[pl.ds(s * RPS, RPS)], acc_sh.at[pl.ds(s * RPS, RPS)])
    pltpu.sync_copy(src_hbm.at[pl.ds(base, CPW)], sidx_v)
    pltpu.sync_copy(dst_hbm.at[pl.ds(base, CPW)], didx_v)
    plsc.subcore_barrier()

    def prime(j, carry):
        @pl.when(j < nj)
        def _():
            pltpu.async_copy(p_hbm.at[sidx_v.at[j + o]], rows_v.at[j], gsem.at[j])
        return carry

    lax.fori_loop(0, PF, prime, 0)

    def body(j, carry):
        b = lax.rem(j, RING)
        bn = lax.rem(j + PF, RING)

        @pl.when(j >= RING - PF)
        def _():
            pltpu.make_async_copy(rows_v.at[bn],
                                  acc_sh.at[didx_v.at[j - (RING - PF) + o]],
                                  ssem.at[bn]).wait()

        @pl.when(j + PF < nj)
        def _():
            pltpu.async_copy(p_hbm.at[sidx_v.at[j + PF + o]], rows_v.at[bn], gsem.at[bn])

        pltpu.make_async_copy(p_hbm.at[sidx_v.at[j + o]], rows_v.at[b], gsem.at[b]).wait()
        pltpu.async_copy(rows_v.at[b], acc_sh.at[didx_v.at[j + o]], ssem.at[b], add=True)
        return carry

    lax.fori_loop(0, nj, body, 0)

    def drain(k, carry):
        j = nj - (RING - PF) + k

        @pl.when(j >= 0)
        def _():
            pltpu.make_async_copy(rows_v.at[lax.rem(j, RING)],
                                  acc_sh.at[didx_v.at[j + o]],
                                  ssem.at[lax.rem(j, RING)]).wait()
        return carry

    lax.fori_loop(0, RING - PF, drain, 0)
    plsc.subcore_barrier()
    pltpu.sync_copy(acc_sh.at[pl.ds(s * RPS, RPS)], out_hbm.at[c, pl.ds(s * RPS, RPS)])


_sc_deg = pl.kernel(
    _sc_deg_body,
    out_type=jax.ShapeDtypeStruct((NC, NP, F), jnp.float32),
    mesh=_mesh,
    compiler_params=_sc_params,
    scratch_types=[
        pltpu.VMEM((B, F), jnp.float32),
        pltpu.VMEM((CPW, B), jnp.int32),
        pltpu.SemaphoreType.DMA,
        pltpu.VMEM_SHARED((NP, F), jnp.float32),
    ],
)

_sc_prop = pl.kernel(
    _sc_prop_body,
    out_type=jax.ShapeDtypeStruct((NC, NP, F), jnp.float32),
    mesh=_mesh,
    compiler_params=_sc_params,
    scratch_types=[
        pltpu.VMEM((CPW, B), jnp.int32),
        pltpu.VMEM((CPW, B), jnp.int32),
        pltpu.VMEM((RING, B, F), jnp.float32),
        pltpu.SemaphoreType.DMA((RING,)),
        pltpu.SemaphoreType.DMA((RING,)),
        pltpu.VMEM_SHARED((NP, F), jnp.float32),
    ],
)


PK = N // 8
PKP = NP // 8


def _tc_cvt_body(ei_ref, s_ref, d_ref):
    s_ref[...] = ei_ref[0]
    d_ref[...] = ei_ref[1]


def _tc_cvt(edge_index):
    return pl.pallas_call(
        _tc_cvt_body,
        out_shape=[jax.ShapeDtypeStruct((E,), jnp.int32),
                   jax.ShapeDtypeStruct((E,), jnp.int32)],
    )(edge_index)


def _tc_mm_body(x_ref, w_ref, p0_ref):
    p0_ref[...] = jnp.dot(x_ref[...], w_ref[...],
                          preferred_element_type=jnp.float32)


def _tc_scale_body(p0_ref, degp_ref, p_ref, dinv_ref):
    deg = degp_ref[0, :PK] + degp_ref[1, :PK]
    dinv = lax.rsqrt(jnp.maximum(deg, 1.0))
    p_ref[...] = p0_ref[...] * dinv
    dinv_ref[...] = dinv


def _tc_b_body(a1p_ref, dinv_ref, q_ref):
    dinv = dinv_ref[...]
    a1 = (a1p_ref[0, :PK] + a1p_ref[1, :PK]) * dinv
    q_ref[...] = jnp.maximum(a1, 0.0) * dinv


def _tc_c_body(a2p_ref, dinv_ref, w2bd_ref, out_ref):
    a2 = (a2p_ref[0, :PK] + a2p_ref[1, :PK]) * dinv_ref[...]
    out_ref[...] = jnp.dot(a2, w2bd_ref[...], preferred_element_type=jnp.float32)


def _tc_mm(x, W1):
    return pl.pallas_call(
        _tc_mm_body,
        out_shape=jax.ShapeDtypeStruct((N, F), jnp.float32),
    )(x, W1)


def _tc_scale(p0, degp_pk):
    return pl.pallas_call(
        _tc_scale_body,
        out_shape=[
            jax.ShapeDtypeStruct((PK, 128), jnp.float32),
            jax.ShapeDtypeStruct((PK, 128), jnp.float32),
        ],
    )(p0, degp_pk)


def _tc_b(a1p_pk, dinv_pk):
    return pl.pallas_call(
        _tc_b_body,
        out_shape=jax.ShapeDtypeStruct((PK, 128), jnp.float32),
    )(a1p_pk, dinv_pk)


def _tc_c(a2p_pk, dinv_pk, W2bd):
    return pl.pallas_call(
        _tc_c_body,
        out_shape=jax.ShapeDtypeStruct((PK, 8 * 64), jnp.float32),
    )(a2p_pk, dinv_pk, W2bd)


@jax.jit
def kernel(x, edge_index, W1, W2):
    src1, dst1 = _tc_cvt(edge_index)
    src2 = src1.reshape(NCHUNK, B)
    dst2 = dst1.reshape(NCHUNK, B)
    zeros_np = jnp.zeros((NP, F), jnp.float32)
    ones_b = jnp.ones((B, F), jnp.float32)

    W2bd = jnp.kron(jnp.eye(8, dtype=jnp.float32), W2)
    p0 = _tc_mm(x, W1)
    deg_parts = _sc_deg(dst2, zeros_np, ones_b)
    p_pk, dinv_pk = _tc_scale(p0.reshape(PK, 128), deg_parts.reshape(NC, PKP, 128))
    a1p = _sc_prop(p_pk.reshape(N, F), src2, dst2, zeros_np)
    q_pk = _tc_b(a1p.reshape(NC, PKP, 128), dinv_pk)
    a2p = _sc_prop(q_pk.reshape(N, F), src2, dst2, zeros_np)
    out = _tc_c(a2p.reshape(NC, PKP, 128), dinv_pk, W2bd)
    return out.reshape(N, 64)


# speedup vs baseline: 1.1337x; 1.0505x over previous
import jax
import jax.numpy as jnp
from jax import lax
from jax.experimental import pallas as pl
from jax.experimental.pallas import tpu as pltpu
from jax.experimental.pallas import tpu_sc as plsc

N = 10000
E = 320000
NP = 10240
F = 16
B = 128
NC = 2
NS = 16
NW = NC * NS
NCHUNK = E // B
CPW = 79
RPS = NP // NS

_mesh = plsc.VectorSubcoreMesh(core_axis_name="c", subcore_axis_name="s")
_sc_params = pltpu.CompilerParams(use_tc_tiling_on_sc=False)


def _sc_deg_body(dst_hbm, zeros_hbm, ones_hbm, out_hbm, ones_v, didx_v, sem, acc_sh):
    c = lax.axis_index("c")
    s = lax.axis_index("s")
    w = s * NC + c
    w79 = w * CPW
    base = jnp.minimum(w79, NCHUNK - CPW)
    o = w79 - base
    nj = jnp.minimum(CPW, NCHUNK - w79)
    pltpu.sync_copy(zeros_hbm.at[pl.ds(s * RPS, RPS)], acc_sh.at[pl.ds(s * RPS, RPS)])
    pltpu.sync_copy(ones_hbm, ones_v)
    pltpu.sync_copy(dst_hbm.at[pl.ds(base, CPW)], didx_v)
    plsc.subcore_barrier()

    def fire(j, carry):
        pltpu.async_copy(ones_v, acc_sh.at[didx_v.at[j + o]], sem, add=True)
        return carry

    lax.fori_loop(0, nj, fire, 0)

    def drain(j, carry):
        pltpu.make_async_copy(ones_v, acc_sh.at[didx_v.at[j + o]], sem).wait()
        return carry

    lax.fori_loop(0, nj, drain, 0)
    plsc.subcore_barrier()
    pltpu.sync_copy(acc_sh.at[pl.ds(s * RPS, RPS)], out_hbm.at[c, pl.ds(s * RPS, RPS)])


RING = 12
PF = 6


def _sc_prop_body(p_hbm, src_hbm, dst_hbm, zeros_hbm, out_hbm,
                  sidx_v, didx_v, rows_v, gsem, ssem, acc_sh):
    c = lax.axis_index("c")
    s = lax.axis_index("s")
    w = s * NC + c
    w79 = w * CPW
    base = jnp.minimum(w79, NCHUNK - CPW)
    o = w79 - base
    nj = jnp.minimum(CPW, NCHUNK - w79)
    pltpu.sync_copy(zeros_hbm.at[pl.ds(s * RPS, RPS)], acc_sh.at[pl.ds(s * RPS, RPS)])
    pltpu.sync_copy(src_hbm.at[pl.ds(base, CPW)], sidx_v)
    pltpu.sync_copy(dst_hbm.at[pl.ds(base, CPW)], didx_v)
    plsc.subcore_barrier()

    def prime(j, carry):
        @pl.when(j < nj)
        def _():
            pltpu.async_copy(p_hbm.at[sidx_v.at[j + o]], rows_v.at[j], gsem.at[j])
        return carry

    lax.fori_loop(0, PF, prime, 0)

    def body(j, carry):
        b = lax.rem(j, RING)
        bn = lax.rem(j + PF, RING)

        @pl.when(j >= RING - PF)
        def _():
            pltpu.make_async_copy(rows_v.at[bn],
                                  acc_sh.at[didx_v.at[j - (RING - PF) + o]],
                                  ssem.at[bn]).wait()

        @pl.when(j + PF < nj)
        def _():
            pltpu.async_copy(p_hbm.at[sidx_v.at[j + PF + o]], rows_v.at[bn], gsem.at[bn])

        pltpu.make_async_copy(p_hbm.at[sidx_v.at[j + o]], rows_v.at[b], gsem.at[b]).wait()
        pltpu.async_copy(rows_v.at[b], acc_sh.at[didx_v.at[j + o]], ssem.at[b], add=True)
        return carry

    lax.fori_loop(0, nj, body, 0)

    def drain(k, carry):
        j = nj - (RING - PF) + k

        @pl.when(j >= 0)
        def _():
            pltpu.make_async_copy(rows_v.at[lax.rem(j, RING)],
                                  acc_sh.at[didx_v.at[j + o]],
                                  ssem.at[lax.rem(j, RING)]).wait()
        return carry

    lax.fori_loop(0, RING - PF, drain, 0)
    plsc.subcore_barrier()
    pltpu.sync_copy(acc_sh.at[pl.ds(s * RPS, RPS)], out_hbm.at[c, pl.ds(s * RPS, RPS)])


_sc_deg = pl.kernel(
    _sc_deg_body,
    out_type=jax.ShapeDtypeStruct((NC, NP, F), jnp.float32),
    mesh=_mesh,
    compiler_params=_sc_params,
    scratch_types=[
        pltpu.VMEM((B, F), jnp.float32),
        pltpu.VMEM((CPW, B), jnp.int32),
        pltpu.SemaphoreType.DMA,
        pltpu.VMEM_SHARED((NP, F), jnp.float32),
    ],
)

_sc_prop = pl.kernel(
    _sc_prop_body,
    out_type=jax.ShapeDtypeStruct((NC, NP, F), jnp.float32),
    mesh=_mesh,
    compiler_params=_sc_params,
    scratch_types=[
        pltpu.VMEM((CPW, B), jnp.int32),
        pltpu.VMEM((CPW, B), jnp.int32),
        pltpu.VMEM((RING, B, F), jnp.float32),
        pltpu.SemaphoreType.DMA((RING,)),
        pltpu.SemaphoreType.DMA((RING,)),
        pltpu.VMEM_SHARED((NP, F), jnp.float32),
    ],
)


PK = N // 8
PKP = NP // 8


def _tc_cvt_body(ei_ref, s_ref, d_ref):
    s_ref[...] = ei_ref[0]
    d_ref[...] = ei_ref[1]


def _tc_cvt(edge_index):
    return pl.pallas_call(
        _tc_cvt_body,
        out_shape=[jax.ShapeDtypeStruct((E,), jnp.int32),
                   jax.ShapeDtypeStruct((E,), jnp.int32)],
    )(edge_index)


def _tc_mm_body(x_ref, w_ref, p0_ref):
    p0_ref[...] = jnp.dot(x_ref[...], w_ref[...],
                          preferred_element_type=jnp.float32)


def _tc_scale_body(p0_ref, degp_ref, p_ref, dinv_ref):
    deg = degp_ref[0, :PK] + degp_ref[1, :PK]
    dinv = lax.rsqrt(jnp.maximum(deg, 1.0))
    p_ref[...] = p0_ref[...] * dinv
    dinv_ref[...] = dinv


def _tc_b_body(a1p_ref, dinv_ref, q_ref):
    dinv = dinv_ref[...]
    a1 = (a1p_ref[0, :PK] + a1p_ref[1, :PK]) * dinv
    q_ref[...] = jnp.maximum(a1, 0.0) * dinv


def _tc_c_body(a2p_ref, dinv_ref, w2bd_ref, out_ref):
    a2 = (a2p_ref[0, :PK] + a2p_ref[1, :PK]) * dinv_ref[...]
    out_ref[...] = jnp.dot(a2, w2bd_ref[...], preferred_element_type=jnp.float32)


def _tc_mm(x, W1):
    return pl.pallas_call(
        _tc_mm_body,
        out_shape=jax.ShapeDtypeStruct((N, F), jnp.float32),
    )(x, W1)


def _tc_scale(p0, degp_pk):
    return pl.pallas_call(
        _tc_scale_body,
        out_shape=[
            jax.ShapeDtypeStruct((PK, 128), jnp.float32),
            jax.ShapeDtypeStruct((PK, 128), jnp.float32),
        ],
    )(p0, degp_pk)


def _tc_b(a1p_pk, dinv_pk):
    return pl.pallas_call(
        _tc_b_body,
        out_shape=jax.ShapeDtypeStruct((PK, 128), jnp.float32),
    )(a1p_pk, dinv_pk)


def _tc_c(a2p_pk, dinv_pk, W2bd):
    return pl.pallas_call(
        _tc_c_body,
        out_shape=jax.ShapeDtypeStruct((PK, 8 * 64), jnp.float32),
    )(a2p_pk, dinv_pk, W2bd)


@jax.jit
def kernel(x, edge_index, W1, W2):
    src1, dst1 = _tc_cvt(edge_index)
    src2 = src1.reshape(NCHUNK, B)
    dst2 = dst1.reshape(NCHUNK, B)
    zeros_np = jnp.zeros((NP, F), jnp.float32)
    ones_b = jnp.ones((B, F), jnp.float32)

    W2bd = jnp.kron(jnp.eye(8, dtype=jnp.float32), W2)
    p0 = _tc_mm(x, W1)
    deg_parts = _sc_deg(dst2, zeros_np, ones_b)
    p_pk, dinv_pk = _tc_scale(p0.reshape(PK, 128), deg_parts.reshape(NC, PKP, 128))
    a1p = _sc_prop(p_pk.reshape(N, F), src2, dst2, zeros_np)
    q_pk = _tc_b(a1p.reshape(NC, PKP, 128), dinv_pk)
    a2p = _sc_prop(q_pk.reshape(N, F), src2, dst2, zeros_np)
    out = _tc_c(a2p.reshape(NC, PKP, 128), dinv_pk, W2bd)
    return out.reshape(N, 64)
